# initial kernel scaffold (unmeasured)
import jax
import jax.numpy as jnp
from jax import lax
from jax.experimental import pallas as pl
from jax.experimental.pallas import tpu as pltpu

N_DEV = 16
M_PER = 256
K = 4096
N_PER = 512


def kernel(x, w_mat):
    def body(x_ref, w_ref, out_ref, gather_ref, w_bf_ref, send_sems, recv_sems):
        my = lax.axis_index("i")
        left = (my - 1) % N_DEV
        right = (my + 1) % N_DEV

        barrier_sem = pltpu.get_barrier_semaphore()
        for nbr in (left, right):
            pl.semaphore_signal(
                barrier_sem, inc=1,
                device_id=(nbr,), device_id_type=pl.DeviceIdType.MESH,
            )
        pl.semaphore_wait(barrier_sem, 2)

        gather_ref[0, :, :] = x_ref[:, :].astype(jnp.bfloat16)
        w_bf_ref[:, :] = w_ref[:, :].astype(jnp.bfloat16)

        def compute(slot):
            origin = (my - slot) % N_DEV
            acc = jnp.dot(
                gather_ref[slot, :, :], w_bf_ref[:, :],
                preferred_element_type=jnp.float32,
            )
            out_ref[pl.ds(origin * M_PER, M_PER), :] = jnp.maximum(acc, 0.0)

        compute(0)

        for h in range(N_DEV - 1):
            send = pltpu.make_async_remote_copy(
                src_ref=gather_ref.at[h],
                dst_ref=gather_ref.at[h + 1],
                send_sem=send_sems.at[h],
                recv_sem=recv_sems.at[h],
                device_id=(right,),
                device_id_type=pl.DeviceIdType.MESH,
            )
            send.start()
            recv = pltpu.make_async_remote_copy(
                src_ref=gather_ref.at[h],
                dst_ref=gather_ref.at[h + 1],
                send_sem=send_sems.at[h],
                recv_sem=recv_sems.at[h],
                device_id=(left,),
                device_id_type=pl.DeviceIdType.MESH,
            )
            recv.wait_recv()
            compute(h + 1)
            send.wait_send()

    out_shape = jax.ShapeDtypeStruct((N_DEV * M_PER, N_PER), jnp.float32)
    return pl.pallas_call(
        body,
        out_shape=out_shape,
        in_specs=[
            pl.BlockSpec(memory_space=pltpu.VMEM),
            pl.BlockSpec(memory_space=pltpu.VMEM),
        ],
        out_specs=pl.BlockSpec(memory_space=pltpu.VMEM),
        scratch_shapes=[
            pltpu.VMEM((N_DEV, M_PER, K), jnp.bfloat16),
            pltpu.VMEM((K, N_PER), jnp.bfloat16),
            pltpu.SemaphoreType.DMA((N_DEV - 1,)),
            pltpu.SemaphoreType.DMA((N_DEV - 1,)),
        ],
        compiler_params=pltpu.CompilerParams(collective_id=0),
    )(x, w_mat)


# baseline (device time: 405077 ns/iter reference)
import jax
import jax.numpy as jnp
from jax import lax
from jax.experimental import pallas as pl
from jax.experimental.pallas import tpu as pltpu

N_DEV = 16
M_PER = 256
K = 4096
N_PER = 512


def kernel(x, w_mat):
    def body(x_ref, w_ref, out_ref, gather_ref, w_bf_ref, send_sems, recv_sems):
        my = lax.axis_index("i")
        left = (my - 1) % N_DEV
        right = (my + 1) % N_DEV

        barrier_sem = pltpu.get_barrier_semaphore()
        for nbr in (left, right):
            pl.semaphore_signal(
                barrier_sem, inc=1,
                device_id=(nbr,), device_id_type=pl.DeviceIdType.MESH,
            )
        pl.semaphore_wait(barrier_sem, 2)

        gather_ref[0, :, :] = x_ref[:, :].astype(jnp.bfloat16)
        w_bf_ref[:, :] = w_ref[:, :].astype(jnp.bfloat16)

        def compute(slot):
            origin = (my - slot) % N_DEV
            acc = jnp.dot(
                gather_ref[slot, :, :], w_bf_ref[:, :],
                preferred_element_type=jnp.float32,
            )
            out_ref[pl.ds(origin * M_PER, M_PER), :] = jnp.maximum(acc, 0.0)

        compute(0)

        for h in range(N_DEV - 1):
            send = pltpu.make_async_remote_copy(
                src_ref=gather_ref.at[h],
                dst_ref=gather_ref.at[h + 1],
                send_sem=send_sems.at[h],
                recv_sem=recv_sems.at[h],
                device_id=(right,),
                device_id_type=pl.DeviceIdType.MESH,
            )
            send.start()
            recv = pltpu.make_async_remote_copy(
                src_ref=gather_ref.at[h],
                dst_ref=gather_ref.at[h + 1],
                send_sem=send_sems.at[h],
                recv_sem=recv_sems.at[h],
                device_id=(left,),
                device_id_type=pl.DeviceIdType.MESH,
            )
            recv.wait_recv()
            compute(h + 1)
            send.wait_send()

    out_shape = jax.ShapeDtypeStruct((N_DEV * M_PER, N_PER), jnp.float32)
    return pl.pallas_call(
        body,
        out_shape=out_shape,
        in_specs=[
            pl.BlockSpec(memory_space=pltpu.VMEM),
            pl.BlockSpec(memory_space=pltpu.VMEM),
        ],
        out_specs=pl.BlockSpec(memory_space=pltpu.VMEM),
        scratch_shapes=[
            pltpu.VMEM((N_DEV, M_PER, K), jnp.bfloat16),
            pltpu.VMEM((K, N_PER), jnp.bfloat16),
            pltpu.SemaphoreType.DMA((N_DEV - 1,)),
            pltpu.SemaphoreType.DMA((N_DEV - 1,)),
        ],
        compiler_params=pltpu.CompilerParams(
            collective_id=0,
            vmem_limit_bytes=100 * 1024 * 1024,
        ),
    )(x, w_mat)


# device time: 213951 ns/iter; 1.8933x vs baseline; 1.8933x over previous
import jax
import jax.numpy as jnp
from jax import lax
from jax.experimental import pallas as pl
from jax.experimental.pallas import tpu as pltpu

N_DEV = 16
M_PER = 256
K = 4096
N_PER = 512

CW_HOPS = N_DEV // 2
CCW_HOPS = N_DEV - 1 - CW_HOPS


def kernel(x, w_mat):
    def body(x_ref, w_ref, out_ref, gather_ref, w_bf_ref,
             cw_send_sems, cw_recv_sems, ccw_send_sems, ccw_recv_sems):
        my = lax.axis_index("i")
        left = (my - 1) % N_DEV
        right = (my + 1) % N_DEV

        barrier_sem = pltpu.get_barrier_semaphore()
        for nbr in (left, right):
            pl.semaphore_signal(
                barrier_sem, inc=1,
                device_id=(nbr,), device_id_type=pl.DeviceIdType.MESH,
            )
        pl.semaphore_wait(barrier_sem, 2)

        gather_ref[0, :, :] = x_ref[:, :].astype(jnp.bfloat16)
        w_bf_ref[:, :] = w_ref[:, :].astype(jnp.bfloat16)

        def cw_send(h):
            d = pltpu.make_async_remote_copy(
                src_ref=gather_ref.at[h],
                dst_ref=gather_ref.at[h + 1],
                send_sem=cw_send_sems.at[h],
                recv_sem=cw_recv_sems.at[h],
                device_id=(right,),
                device_id_type=pl.DeviceIdType.MESH,
            )
            d.start()
            return d

        def ccw_send(h):
            d = pltpu.make_async_remote_copy(
                src_ref=gather_ref.at[(N_DEV - h) % N_DEV],
                dst_ref=gather_ref.at[N_DEV - 1 - h],
                send_sem=ccw_send_sems.at[h],
                recv_sem=ccw_recv_sems.at[h],
                device_id=(left,),
                device_id_type=pl.DeviceIdType.MESH,
            )
            d.start()
            return d

        def cw_recv(h):
            return pltpu.make_async_remote_copy(
                src_ref=gather_ref.at[h],
                dst_ref=gather_ref.at[h + 1],
                send_sem=cw_send_sems.at[h],
                recv_sem=cw_recv_sems.at[h],
                device_id=(left,),
                device_id_type=pl.DeviceIdType.MESH,
            )

        def ccw_recv(h):
            return pltpu.make_async_remote_copy(
                src_ref=gather_ref.at[(N_DEV - h) % N_DEV],
                dst_ref=gather_ref.at[N_DEV - 1 - h],
                send_sem=ccw_send_sems.at[h],
                recv_sem=ccw_recv_sems.at[h],
                device_id=(right,),
                device_id_type=pl.DeviceIdType.MESH,
            )

        def compute(slot):
            origin = (my - slot) % N_DEV
            acc = jnp.dot(
                gather_ref[slot, :, :], w_bf_ref[:, :],
                preferred_element_type=jnp.float32,
            )
            out_ref[pl.ds(origin * M_PER, M_PER), :] = jnp.maximum(acc, 0.0)

        sends = [cw_send(0), ccw_send(0)]
        compute(0)

        for h in range(CW_HOPS):
            cw_recv(h).wait_recv()
            if h + 1 < CW_HOPS:
                sends.append(cw_send(h + 1))
            if h < CCW_HOPS:
                ccw_recv(h).wait_recv()
                if h + 1 < CCW_HOPS:
                    sends.append(ccw_send(h + 1))
                compute(h + 1)
                compute(N_DEV - 1 - h)
            else:
                compute(h + 1)

        for d in sends:
            d.wait_send()

    out_shape = jax.ShapeDtypeStruct((N_DEV * M_PER, N_PER), jnp.float32)
    return pl.pallas_call(
        body,
        out_shape=out_shape,
        in_specs=[
            pl.BlockSpec(memory_space=pltpu.VMEM),
            pl.BlockSpec(memory_space=pltpu.VMEM),
        ],
        out_specs=pl.BlockSpec(memory_space=pltpu.VMEM),
        scratch_shapes=[
            pltpu.VMEM((N_DEV, M_PER, K), jnp.bfloat16),
            pltpu.VMEM((K, N_PER), jnp.bfloat16),
            pltpu.SemaphoreType.DMA((CW_HOPS,)),
            pltpu.SemaphoreType.DMA((CW_HOPS,)),
            pltpu.SemaphoreType.DMA((CCW_HOPS,)),
            pltpu.SemaphoreType.DMA((CCW_HOPS,)),
        ],
        compiler_params=pltpu.CompilerParams(
            collective_id=0,
            vmem_limit_bytes=100 * 1024 * 1024,
        ),
    )(x, w_mat)


# device time: 204595 ns/iter; 1.9799x vs baseline; 1.0457x over previous
import jax
import jax.numpy as jnp
from jax import lax
from jax.experimental import pallas as pl
from jax.experimental.pallas import tpu as pltpu

N_DEV = 16
M_PER = 256
M_HALF = M_PER // 2
K = 4096
N_PER = 512

HOPS = N_DEV // 2


def kernel(x, w_mat):
    def body(x_ref, w_ref, out_ref, gather_ref, w_bf_ref,
             cw_send_sems, cw_recv_sems, ccw_send_sems, ccw_recv_sems):
        my = lax.axis_index("i")
        left = (my - 1) % N_DEV
        right = (my + 1) % N_DEV

        barrier_sem = pltpu.get_barrier_semaphore()
        for nbr in (left, right):
            pl.semaphore_signal(
                barrier_sem, inc=1,
                device_id=(nbr,), device_id_type=pl.DeviceIdType.MESH,
            )
        pl.semaphore_wait(barrier_sem, 2)

        def _rdma(src, dst, ssem, rsem, target):
            return pltpu.make_async_remote_copy(
                src_ref=src, dst_ref=dst, send_sem=ssem, recv_sem=rsem,
                device_id=(target,), device_id_type=pl.DeviceIdType.MESH,
            )

        def cw_desc(h, target):
            if h < HOPS - 1:
                src, dst = gather_ref.at[h], gather_ref.at[h + 1]
            else:
                src = gather_ref.at[HOPS - 1, pl.ds(0, M_HALF), :]
                dst = gather_ref.at[HOPS, pl.ds(0, M_HALF), :]
            return _rdma(src, dst, cw_send_sems.at[h], cw_recv_sems.at[h],
                         target)

        def ccw_desc(h, target):
            if h < HOPS - 1:
                src = gather_ref.at[(N_DEV - h) % N_DEV]
                dst = gather_ref.at[N_DEV - 1 - h]
            else:
                src = gather_ref.at[HOPS + 1, pl.ds(M_HALF, M_HALF), :]
                dst = gather_ref.at[HOPS, pl.ds(M_HALF, M_HALF), :]
            return _rdma(src, dst, ccw_send_sems.at[h], ccw_recv_sems.at[h],
                         target)

        def compute(slot):
            origin = (my - slot) % N_DEV
            acc = jnp.dot(
                gather_ref[slot, :, :], w_bf_ref[:, :],
                preferred_element_type=jnp.float32,
            )
            out_ref[pl.ds(origin * M_PER, M_PER), :] = jnp.maximum(acc, 0.0)

        gather_ref[0, :, :] = x_ref[:, :].astype(jnp.bfloat16)
        sends = [cw_desc(0, right), ccw_desc(0, left)]
        sends[0].start()
        sends[1].start()
        w_bf_ref[:, :] = w_ref[:, :].astype(jnp.bfloat16)
        compute(0)

        for h in range(HOPS - 1):
            cw_desc(h, left).wait_recv()
            nxt = cw_desc(h + 1, right)
            nxt.start()
            sends.append(nxt)
            ccw_desc(h, right).wait_recv()
            nxt = ccw_desc(h + 1, left)
            nxt.start()
            sends.append(nxt)
            compute(h + 1)
            compute(N_DEV - 1 - h)

        cw_desc(HOPS - 1, left).wait_recv()
        ccw_desc(HOPS - 1, right).wait_recv()
        compute(HOPS)

        for d in sends:
            d.wait_send()

    out_shape = jax.ShapeDtypeStruct((N_DEV * M_PER, N_PER), jnp.float32)
    return pl.pallas_call(
        body,
        out_shape=out_shape,
        in_specs=[
            pl.BlockSpec(memory_space=pltpu.VMEM),
            pl.BlockSpec(memory_space=pltpu.VMEM),
        ],
        out_specs=pl.BlockSpec(memory_space=pltpu.VMEM),
        scratch_shapes=[
            pltpu.VMEM((N_DEV, M_PER, K), jnp.bfloat16),
            pltpu.VMEM((K, N_PER), jnp.bfloat16),
            pltpu.SemaphoreType.DMA((HOPS,)),
            pltpu.SemaphoreType.DMA((HOPS,)),
            pltpu.SemaphoreType.DMA((HOPS,)),
            pltpu.SemaphoreType.DMA((HOPS,)),
        ],
        compiler_params=pltpu.CompilerParams(
            collective_id=0,
            vmem_limit_bytes=100 * 1024 * 1024,
        ),
    )(x, w_mat)


# device time: 189853 ns/iter; 2.1336x vs baseline; 1.0776x over previous
import jax
import jax.numpy as jnp
from jax import lax
from jax.experimental import pallas as pl
from jax.experimental.pallas import tpu as pltpu

N_DEV = 16
M_PER = 256
M_HALF = M_PER // 2
K = 4096
N_PER = 512

N_MSG = N_DEV - 1

PERM = [0, 4, 8, 12, 15, 11, 7, 3, 2, 6, 10, 14, 13, 9, 5, 1]
INV = [0] * N_DEV
for _k, _l in enumerate(PERM):
    INV[_l] = _k
RIGHT = [PERM[(INV[_l] + 1) % N_DEV] for _l in range(N_DEV)]
LEFT = [PERM[(INV[_l] - 1) % N_DEV] for _l in range(N_DEV)]


def _lut(table, idx):
    r = jnp.int32(table[0])
    for v in range(1, len(table)):
        r = jnp.where(idx == v, jnp.int32(table[v]), r)
    return r


def kernel(x, w_mat):
    def body(x_ref, w_ref, out_ref, gather_ref, w_bf_ref,
             cw_send_sems, cw_recv_sems, ccw_send_sems, ccw_recv_sems):
        my = lax.axis_index("i")
        pos = _lut(INV, my)
        right = _lut(RIGHT, my)
        left = _lut(LEFT, my)

        barrier_sem = pltpu.get_barrier_semaphore()
        for nbr in (left, right):
            pl.semaphore_signal(
                barrier_sem, inc=1,
                device_id=(nbr,), device_id_type=pl.DeviceIdType.MESH,
            )
        pl.semaphore_wait(barrier_sem, 2)

        def _rdma(src, dst, ssem, rsem, target):
            return pltpu.make_async_remote_copy(
                src_ref=src, dst_ref=dst, send_sem=ssem, recv_sem=rsem,
                device_id=(target,), device_id_type=pl.DeviceIdType.MESH,
            )

        def cw_desc(j, target):
            s, h = j // 2, j % 2
            return _rdma(
                gather_ref.at[s, pl.ds(h * M_HALF, M_HALF), :],
                gather_ref.at[s + 1, pl.ds(h * M_HALF, M_HALF), :],
                cw_send_sems.at[j], cw_recv_sems.at[j], target,
            )

        def ccw_desc(j, target):
            s, h = j // 2, 1 - (j % 2)
            return _rdma(
                gather_ref.at[(N_DEV - s) % N_DEV, pl.ds(h * M_HALF, M_HALF), :],
                gather_ref.at[N_DEV - 1 - s, pl.ds(h * M_HALF, M_HALF), :],
                ccw_send_sems.at[j], ccw_recv_sems.at[j], target,
            )

        def compute_half(slot, half):
            origin = _lut(PERM, (pos - slot) % N_DEV)
            acc = jnp.dot(
                gather_ref[slot, pl.ds(half * M_HALF, M_HALF), :],
                w_bf_ref[:, :],
                preferred_element_type=jnp.float32,
            )
            row = origin * M_PER + half * M_HALF
            out_ref[pl.ds(row, M_HALF), :] = jnp.maximum(acc, 0.0)

        gather_ref[0, :, :] = x_ref[:, :].astype(jnp.bfloat16)
        sends = []
        for j in (0, 1):
            d = cw_desc(j, right)
            d.start()
            sends.append(d)
            d = ccw_desc(j, left)
            d.start()
            sends.append(d)
        w_bf_ref[:, :] = w_ref[:, :].astype(jnp.bfloat16)
        compute_half(0, 0)
        compute_half(0, 1)

        for j in range(N_MSG):
            cw_desc(j, left).wait_recv()
            if j + 2 < N_MSG:
                d = cw_desc(j + 2, right)
                d.start()
                sends.append(d)
            ccw_desc(j, right).wait_recv()
            if j + 2 < N_MSG:
                d = ccw_desc(j + 2, left)
                d.start()
                sends.append(d)
            compute_half(j // 2 + 1, j % 2)
            compute_half(N_DEV - 1 - j // 2, 1 - (j % 2))

        for d in sends:
            d.wait_send()

    out_shape = jax.ShapeDtypeStruct((N_DEV * M_PER, N_PER), jnp.float32)
    return pl.pallas_call(
        body,
        out_shape=out_shape,
        in_specs=[
            pl.BlockSpec(memory_space=pltpu.VMEM),
            pl.BlockSpec(memory_space=pltpu.VMEM),
        ],
        out_specs=pl.BlockSpec(memory_space=pltpu.VMEM),
        scratch_shapes=[
            pltpu.VMEM((N_DEV, M_PER, K), jnp.bfloat16),
            pltpu.VMEM((K, N_PER), jnp.bfloat16),
            pltpu.SemaphoreType.DMA((N_MSG,)),
            pltpu.SemaphoreType.DMA((N_MSG,)),
            pltpu.SemaphoreType.DMA((N_MSG,)),
            pltpu.SemaphoreType.DMA((N_MSG,)),
        ],
        compiler_params=pltpu.CompilerParams(
            collective_id=0,
            vmem_limit_bytes=100 * 1024 * 1024,
        ),
    )(x, w_mat)
